# Initial kernel scaffold; baseline (speedup 1.0000x reference)
import jax
import jax.numpy as jnp
from jax.experimental import pallas as pl


def kernel(x, edge_index, batch, edge_attr, W1, b1, W2, b2, Wl, bl):
    raise NotImplementedError("write your pallas kernel here")


# speedup vs baseline: 3.4430x; 3.4430x over previous
import functools

import jax
import jax.numpy as jnp
from jax import lax
from jax.experimental import pallas as pl
from jax.experimental.pallas import tpu as pltpu
from jax.experimental.pallas import tpu_sc as plsc

N = 10000
E = 320000
H = 128
NG = 64

NC = 2
NS = 16
NW = NC * NS
EW = E // NW
CH = 80
NCHUNK = EW // CH
RPT = 640
RPT_LAST = N - (NS - 1) * RPT
VBUF = 640

def _sc_degree_body(dst_hbm, ew_hbm, out0_hbm, out1_hbm, acc_sh, vbuf, dstv, ewv, sem):
    cid = lax.axis_index("c")
    sid = lax.axis_index("s")
    wid = cid * NS + sid
    r0 = sid * RPT

    def z16(i, c):
        vbuf[pl.ds(i * 16, 16)] = jnp.zeros((16,), jnp.float32)
        return c

    lax.fori_loop(0, VBUF // 16, z16, 0)

    @pl.when(sid < NS - 1)
    def _():
        pltpu.sync_copy(vbuf.at[pl.ds(0, RPT)], acc_sh.at[pl.ds(r0, RPT)])

    @pl.when(sid == NS - 1)
    def _():
        pltpu.sync_copy(vbuf.at[pl.ds(0, RPT_LAST)],
                        acc_sh.at[pl.ds(r0, RPT_LAST)])

    plsc.subcore_barrier()
    ebase = wid * EW

    def chunk(ci, carry):
        base = ebase + ci * CH
        pltpu.sync_copy(dst_hbm.at[pl.ds(base, CH)], dstv)
        pltpu.sync_copy(ew_hbm.at[pl.ds(base, CH)], ewv)
        pltpu.async_copy(ewv, acc_sh.at[dstv], sem, add=True).wait()
        return carry

    lax.fori_loop(0, NCHUNK, chunk, 0)
    plsc.subcore_barrier()

    for c, out_hbm in enumerate((out0_hbm, out1_hbm)):
        @pl.when(jnp.logical_and(cid == c, sid < NS - 1))
        def _(out_hbm=out_hbm):
            pltpu.sync_copy(acc_sh.at[pl.ds(r0, RPT)], vbuf.at[pl.ds(0, RPT)])
            pltpu.sync_copy(vbuf.at[pl.ds(0, RPT)], out_hbm.at[pl.ds(r0, RPT)])

        @pl.when(jnp.logical_and(cid == c, sid == NS - 1))
        def _(out_hbm=out_hbm):
            pltpu.sync_copy(acc_sh.at[pl.ds(r0, RPT_LAST)],
                            vbuf.at[pl.ds(0, RPT_LAST)])
            pltpu.sync_copy(vbuf.at[pl.ds(0, RPT_LAST)],
                            out_hbm.at[pl.ds(r0, RPT_LAST)])


HC = H // NC
EW_T = E // NS
NCHUNK_T = EW_T // CH
ACCW = N * HC
TW = ACCW // NS
ZB = 8000


def _sc_aggregate_body(y_hbm, src_hbm, dst_hbm, ew_hbm, out0_hbm, out1_hbm,
                       acc, rows_v, idxb, valb, vbuf, srcv, dstv, ewv,
                       gsem, ssem):
    cid = lax.axis_index("c")
    sid = lax.axis_index("s")
    col0 = cid * HC
    w0 = sid * TW

    iota16 = lax.iota(jnp.int32, 16)

    def z16(i, c):
        vbuf[pl.ds(i * 16, 16)] = jnp.zeros((16,), jnp.float32)
        return c

    lax.fori_loop(0, ZB // 16, z16, 0)

    def zcp(i, c):
        pltpu.sync_copy(vbuf, acc.at[pl.ds(w0 + i * ZB, ZB)])
        return c

    lax.fori_loop(0, TW // ZB, zcp, 0)
    plsc.subcore_barrier()

    ebase = sid * EW_T

    def chunk(ci, carry):
        base = ebase + ci * CH
        pltpu.sync_copy(src_hbm.at[pl.ds(base, CH)], srcv)
        pltpu.async_copy(y_hbm.at[srcv], rows_v, gsem).wait()
        pltpu.sync_copy(dst_hbm.at[pl.ds(base, CH)], dstv)
        pltpu.sync_copy(ew_hbm.at[pl.ds(base, CH)], ewv)

        def grp(g, c2):
            sl16 = pl.ds(g * 16, 16)
            wv = ewv[sl16]
            dv = dstv[sl16]
            for j in range(16):
                e = g * 16 + j
                w = wv[j]
                d64 = dv[j] * HC
                for k in range(HC // 16):
                    sk = pl.ds(k * 16, 16)
                    idxb[e, sk] = iota16 + (d64 + k * 16)
                    valb[e, sk] = rows_v[e, pl.ds(col0 + k * 16, 16)] * w
            return c2

        lax.fori_loop(0, CH // 16, grp, 0)
        descs = [
            pltpu.async_copy(valb.at[e], acc.at[idxb.at[e]], ssem, add=True)
            for e in range(CH)
        ]
        for de in descs:
            de.wait()
        return carry

    lax.fori_loop(0, NCHUNK_T, chunk, 0)
    plsc.subcore_barrier()

    for c, out_hbm in enumerate((out0_hbm, out1_hbm)):
        @pl.when(cid == c)
        def _(out_hbm=out_hbm):
            def wcp(i, cc):
                off = w0 + i * ZB
                pltpu.sync_copy(acc.at[pl.ds(off, ZB)], vbuf)
                pltpu.sync_copy(vbuf, out_hbm.at[pl.ds(off, ZB)])
                return cc

            lax.fori_loop(0, TW // ZB, wcp, 0)


@functools.lru_cache(maxsize=1)
def _build_sc_kernels():
    mesh = plsc.VectorSubcoreMesh(core_axis_name="c", subcore_axis_name="s")
    sc_degree = pl.kernel(
        _sc_degree_body,
        out_type=[jax.ShapeDtypeStruct((N,), jnp.float32),
                  jax.ShapeDtypeStruct((N,), jnp.float32)],
        mesh=mesh,
        scratch_types=[
            pltpu.VMEM_SHARED((N,), jnp.float32),
            pltpu.VMEM((VBUF,), jnp.float32),
            pltpu.VMEM((CH,), jnp.int32),
            pltpu.VMEM((CH,), jnp.float32),
            pltpu.SemaphoreType.DMA,
        ],
    )
    sc_aggregate = pl.kernel(
        _sc_aggregate_body,
        out_type=[jax.ShapeDtypeStruct((ACCW,), jnp.float32),
                  jax.ShapeDtypeStruct((ACCW,), jnp.float32)],
        mesh=mesh,
        scratch_types=[
            pltpu.VMEM_SHARED((ACCW,), jnp.float32),
            pltpu.VMEM((CH, H), jnp.float32),
            pltpu.VMEM((CH, HC), jnp.int32),
            pltpu.VMEM((CH, HC), jnp.float32),
            pltpu.VMEM((ZB,), jnp.float32),
            pltpu.VMEM((CH,), jnp.int32),
            pltpu.VMEM((CH,), jnp.int32),
            pltpu.VMEM((CH,), jnp.float32),
            pltpu.SemaphoreType.DMA,
            pltpu.SemaphoreType.DMA,
        ],
    )
    return sc_degree, sc_aggregate


_BN = 1000
_GRID = N // _BN

def _dot(a, b):
    return lax.dot_general(
        a.astype(jnp.bfloat16), b.astype(jnp.bfloat16),
        dimension_numbers=(((1,), (0,)), ((), ())),
        preferred_element_type=jnp.float32,
    )


def _tc_a_body(x_ref, w1_ref, degp_ref, y_ref, dinv_ref):
    deg = degp_ref[0] + degp_ref[1] + 1.0
    r = lax.rsqrt(deg)
    dinv = r * (1.5 - 0.5 * deg * r * r)
    y_ref[...] = _dot(x_ref[...], w1_ref[...]) * dinv
    dinv_ref[...] = dinv


def _tc_a(x, W1, degp):
    return pl.pallas_call(
        _tc_a_body,
        grid=(_GRID,),
        in_specs=[
            pl.BlockSpec((_BN, H), lambda i: (i, 0)),
            pl.BlockSpec((H, H), lambda i: (0, 0)),
            pl.BlockSpec((NC, _BN, 1), lambda i: (0, i, 0)),
        ],
        out_specs=[
            pl.BlockSpec((_BN, H), lambda i: (i, 0)),
            pl.BlockSpec((_BN, 1), lambda i: (i, 0)),
        ],
        out_shape=[
            jax.ShapeDtypeStruct((N, H), jnp.float32),
            jax.ShapeDtypeStruct((N, 1), jnp.float32),
        ],
    )(x, W1, degp)


def _tc_b_body(acc_ref, y_ref, dinv_ref, b_ref, w2_ref, y2_ref):
    dinv = dinv_ref[...]
    h = jnp.maximum((acc_ref[...] + y_ref[...]) * dinv + b_ref[...], 0.0)
    y2_ref[...] = _dot(h, w2_ref[...]) * dinv


def _tc_b(acc, y, dinv, b1, W2):
    return pl.pallas_call(
        _tc_b_body,
        grid=(_GRID,),
        in_specs=[
            pl.BlockSpec((_BN, H), lambda i: (i, 0)),
            pl.BlockSpec((_BN, H), lambda i: (i, 0)),
            pl.BlockSpec((_BN, 1), lambda i: (i, 0)),
            pl.BlockSpec((1, H), lambda i: (0, 0)),
            pl.BlockSpec((H, H), lambda i: (0, 0)),
        ],
        out_specs=pl.BlockSpec((_BN, H), lambda i: (i, 0)),
        out_shape=jax.ShapeDtypeStruct((N, H), jnp.float32),
    )(acc, y, dinv, b1, W2)


def _tc_c_body(acc_ref, y_ref, dinv_ref, b_ref, wl_ref, batch_ref, bl_ref,
               out_ref, pool_acc, cnt_acc):
    i = pl.program_id(0)
    h = jnp.maximum(
        (acc_ref[...] + y_ref[...]) * dinv_ref[...] + b_ref[...], 0.0)
    gi = lax.broadcasted_iota(jnp.int32, (_BN, NG), 1)
    onehot = (batch_ref[...] == gi).astype(jnp.float32)
    part = lax.dot_general(
        onehot, h, (((0,), (0,)), ((), ())),
        precision=lax.Precision.HIGHEST,
        preferred_element_type=jnp.float32,
    )
    cpart = lax.dot_general(
        onehot, jnp.ones((_BN, 1), jnp.float32), (((0,), (0,)), ((), ())),
        precision=lax.Precision.HIGHEST,
        preferred_element_type=jnp.float32,
    )

    @pl.when(i == 0)
    def _():
        pool_acc[...] = part
        cnt_acc[...] = cpart

    @pl.when(i != 0)
    def _():
        pool_acc[...] = pool_acc[...] + part
        cnt_acc[...] = cnt_acc[...] + cpart

    @pl.when(i == pl.num_programs(0) - 1)
    def _():
        pooled = pool_acc[...] / jnp.maximum(cnt_acc[...], 1.0)
        out_ref[...] = _dot(pooled, wl_ref[...]) + bl_ref[...]


def _tc_c(acc, y, dinv, b2, Wl, batch2d, bl):
    return pl.pallas_call(
        _tc_c_body,
        grid=(_GRID,),
        in_specs=[
            pl.BlockSpec((_BN, H), lambda i: (i, 0)),
            pl.BlockSpec((_BN, H), lambda i: (i, 0)),
            pl.BlockSpec((_BN, 1), lambda i: (i, 0)),
            pl.BlockSpec((1, H), lambda i: (0, 0)),
            pl.BlockSpec((H, 1), lambda i: (0, 0)),
            pl.BlockSpec((_BN, 1), lambda i: (i, 0)),
            pl.BlockSpec((1, 1), lambda i: (0, 0)),
        ],
        out_specs=pl.BlockSpec((NG, 1), lambda i: (0, 0)),
        out_shape=jax.ShapeDtypeStruct((NG, 1), jnp.float32),
        scratch_shapes=[pltpu.VMEM((NG, H), jnp.float32),
                        pltpu.VMEM((NG, 1), jnp.float32)],
    )(acc, y, dinv, b2, Wl, batch2d, bl)


def kernel(x, edge_index, batch, edge_attr, W1, b1, W2, b2, Wl, bl):
    src = edge_index[0].astype(jnp.int32)
    dst = edge_index[1].astype(jnp.int32)
    ew = edge_attr.astype(jnp.float32)
    batch2d = batch.astype(jnp.int32).reshape(N, 1)

    sc_degree, sc_aggregate = _build_sc_kernels()
    deg0, deg1 = sc_degree(dst, ew)
    degp = jnp.stack([deg0, deg1]).reshape(NC, N, 1)
    y1, dinv = _tc_a(x, W1, degp)
    a10, a11 = sc_aggregate(y1, src, dst, ew)
    acc1 = jnp.concatenate([a10.reshape(N, HC), a11.reshape(N, HC)], axis=1)
    y2 = _tc_b(acc1, y1, dinv, b1.reshape(1, H), W2)
    a20, a21 = sc_aggregate(y2, src, dst, ew)
    acc2 = jnp.concatenate([a20.reshape(N, HC), a21.reshape(N, HC)], axis=1)
    return _tc_c(acc2, y2, dinv, b2.reshape(1, H), Wl, batch2d,
                 bl.reshape(1, 1))
